# baseline (speedup 1.0000x reference)
import functools

import jax
import jax.numpy as jnp
from jax import lax
from jax.experimental import pallas as pl
from jax.experimental.pallas import tpu as pltpu
from jax.experimental.pallas import tpu_sc as plsc

NN = 10000
EE = 320000
DD = 128
DEE = 16
OUTD = 128

NC = 2
NS = 16
NW = NC * NS
NPW = 320
KB = 8000
NBLK = EE // KB
GRP = 96
STG = 80


def _node_pre_body(x_ref, w_ref, b_ref, a_ref, b2_ref, xu_ref):
    abx = (
        jnp.dot(x_ref[...], w_ref[...], preferred_element_type=jnp.float32)
        + b_ref[...]
    )
    a_ref[...] = abx[:, :OUTD]
    b2_ref[...] = abx[:, OUTD:2 * OUTD]
    xu_ref[...] = abx[:, 2 * OUTD:]


def _node_pre(x, Wcat, bcat):
    sds = jax.ShapeDtypeStruct((NN, OUTD), jnp.float32)
    return pl.pallas_call(
        _node_pre_body,
        out_shape=[sds, sds, sds],
    )(x, Wcat, bcat)


def _edge_pre_body(e_ref, w_ref, b_ref, o_ref):
    o_ref[...] = (
        jnp.dot(e_ref[...], w_ref[...], preferred_element_type=jnp.float32)
        + b_ref[...]
    )


def _edge_pre(edge_attr, W3, b_msg):
    RB = 20000
    grid = EE // RB
    return pl.pallas_call(
        _edge_pre_body,
        grid=(grid,),
        in_specs=[
            pl.BlockSpec((RB, DEE), lambda i: (i, 0)),
            pl.BlockSpec((DEE, OUTD), lambda i: (0, 0)),
            pl.BlockSpec((1, OUTD), lambda i: (0, 0)),
        ],
        out_specs=pl.BlockSpec((RB, OUTD), lambda i: (i, 0)),
        out_shape=jax.ShapeDtypeStruct((EE, OUTD), jnp.float32),
    )(edge_attr, W3, b_msg)


def _final_body(s_ref, b_ref, xu_ref, w_ref, o_ref):
    agg = jnp.maximum(s_ref[...] + b_ref[...], 0.0)
    o_ref[...] = xu_ref[...] + jnp.dot(
        agg, w_ref[...], preferred_element_type=jnp.float32
    )


def _final(S, B, XU, Wu2):
    return pl.pallas_call(
        _final_body,
        out_shape=jax.ShapeDtypeStruct((NN, OUTD), jnp.float32),
    )(S, B, XU, Wu2)


def _segmax_body(src_hbm, dst_hbm, a_hbm, c_hbm, s_hbm,
                 acc0, acc1, acc2, acc3, acc4, acc5, acc6, acc7,
                 dblk, sblk, mbuf, sidx, dlbuf, arows, crows, stage,
                 sem_a, sem_c):
    wid = lax.axis_index("s") * NC + lax.axis_index("c")
    lo = wid * NPW
    hi = lo + NPW
    accs = [acc0, acc1, acc2, acc3, acc4, acc5, acc6, acc7]

    ninf = jnp.full((16,), -jnp.inf, jnp.float32)
    zeros = jnp.zeros((16,), jnp.int32)
    ones = jnp.ones((16,), jnp.int32)
    lanes = lax.iota(jnp.int32, 16)

    def _init_acc(r, _):
        for f in range(OUTD // 16):
            accs[f][pl.ds(r * 16, 16)] = ninf
        return 0
    lax.fori_loop(0, NPW + 1, _init_acc, 0)

    def _init_mbuf(k, _):
        mbuf[pl.ds(k * 16, 16)] = zeros
        return 0
    lax.fori_loop(0, (KB + GRP) // 16, _init_mbuf, 0)

    def _block(b, _):
        base = b * KB
        pltpu.sync_copy(dst_hbm.at[pl.ds(base, KB)], dblk)
        pltpu.sync_copy(src_hbm.at[pl.ds(base, KB)], sblk)

        def _scan(i, offv):
            d = dblk[pl.ds(i * 16, 16)]
            m = (d >= lo) & (d < hi)
            mi = jnp.where(m, ones, zeros)
            cntv = plsc.all_reduce_population_count(m)
            pos = offv + plsc.cumsum(mi) - 1
            pos = jnp.where(m, pos, 0)
            eids = base + i * 16 + lanes
            plsc.store_scatter(mbuf, [pos], eids, mask=m)
            return offv + cntv

        offv = lax.fori_loop(0, KB // 16, _scan, zeros, unroll=8)
        m_cnt = offv[0]

        ngrp = (m_cnt + GRP - 1) // GRP

        def _issue(g, par):
            goff = g * GRP
            rem = m_cnt - goff

            def _pre(s, _):
                mv = mbuf[pl.ds(goff + s * 16, 16)]
                valid = (lanes + s * 16) < rem
                lid = jnp.where(valid, mv - base, 0)
                dstv = plsc.load_gather(dblk, [lid])
                srcv = plsc.load_gather(sblk, [lid])
                dlbuf[par, pl.ds(s * 16, 16)] = jnp.where(valid, dstv - lo, NPW)
                sidx[par, pl.ds(s * 16, 16)] = srcv
                return 0
            lax.fori_loop(0, GRP // 16, _pre, 0, unroll=True)

            pltpu.async_copy(
                c_hbm.at[mbuf.at[pl.ds(goff, GRP)]],
                crows.at[pl.ds(par * GRP, GRP)], sem_c.at[par])
            pltpu.async_copy(
                a_hbm.at[sidx.at[par]],
                arows.at[pl.ds(par * GRP, GRP)], sem_a.at[par])

        def _process(g, par):
            goff = g * GRP
            pltpu.make_async_copy(
                a_hbm.at[sidx.at[par]],
                arows.at[pl.ds(par * GRP, GRP)], sem_a.at[par]).wait()
            pltpu.make_async_copy(
                c_hbm.at[mbuf.at[pl.ds(goff, GRP)]],
                crows.at[pl.ds(par * GRP, GRP)], sem_c.at[par]).wait()

            for s in range(GRP // 16):
                dlv = dlbuf[par, pl.ds(s * 16, 16)]
                for j in range(16):
                    e = par * GRP + s * 16 + j
                    dspl = dlv.at[jnp.full((16,), j, jnp.int32)].get(
                        mode="promise_in_bounds")
                    fidx = dspl * 16 + lanes
                    nf = OUTD // 16
                    avs = [arows[e, pl.ds(f * 16, 16)] for f in range(nf)]
                    cvs = [crows[e, pl.ds(f * 16, 16)] for f in range(nf)]
                    gvs = [plsc.load_gather(accs[f], [fidx])
                           for f in range(nf)]
                    nvs = [jnp.maximum(gvs[f], avs[f] + cvs[f])
                           for f in range(nf)]
                    for f in range(nf):
                        plsc.store_scatter(accs[f], [fidx], nvs[f])

        @pl.when(ngrp > 0)
        def _():
            _issue(0, 0)

        def _gloop(g, _):
            par = g % 2

            @pl.when(g + 1 < ngrp)
            def _():
                _issue(g + 1, 1 - par)

            _process(g, par)
            return 0

        lax.fori_loop(0, ngrp, _gloop, 0)
        return 0

    lax.fori_loop(0, NBLK, _block, 0)

    nchunks = jnp.where(wid < NW - 1, NPW // STG, 1)

    def _wchunk(k, _):
        def _wrow(r, _2):
            for f in range(OUTD // 16):
                stage[r, pl.ds(f * 16, 16)] = accs[f][pl.ds((k * STG + r) * 16, 16)]
            return 0
        lax.fori_loop(0, STG, _wrow, 0)
        pltpu.sync_copy(stage, s_hbm.at[pl.ds(lo + k * STG, STG)])
        return 0

    lax.fori_loop(0, nchunks, _wchunk, 0)


def _segmax(src, dst, A, C):
    mesh = plsc.VectorSubcoreMesh(core_axis_name="c", subcore_axis_name="s")
    f = functools.partial(
        pl.kernel,
        out_type=jax.ShapeDtypeStruct((NN, OUTD), jnp.float32),
        mesh=mesh,
        compiler_params=pltpu.CompilerParams(needs_layout_passes=False),
        scratch_types=[
            pltpu.VMEM(((NPW + 1) * 16,), jnp.float32),
            pltpu.VMEM(((NPW + 1) * 16,), jnp.float32),
            pltpu.VMEM(((NPW + 1) * 16,), jnp.float32),
            pltpu.VMEM(((NPW + 1) * 16,), jnp.float32),
            pltpu.VMEM(((NPW + 1) * 16,), jnp.float32),
            pltpu.VMEM(((NPW + 1) * 16,), jnp.float32),
            pltpu.VMEM(((NPW + 1) * 16,), jnp.float32),
            pltpu.VMEM(((NPW + 1) * 16,), jnp.float32),
            pltpu.VMEM((KB,), jnp.int32),
            pltpu.VMEM((KB,), jnp.int32),
            pltpu.VMEM((KB + GRP,), jnp.int32),
            pltpu.VMEM((2, GRP), jnp.int32),
            pltpu.VMEM((2, GRP), jnp.int32),
            pltpu.VMEM((2 * GRP, OUTD), jnp.float32),
            pltpu.VMEM((2 * GRP, OUTD), jnp.float32),
            pltpu.VMEM((STG, OUTD), jnp.float32),
            pltpu.SemaphoreType.DMA((2,)),
            pltpu.SemaphoreType.DMA((2,)),
        ],
    )(_segmax_body)
    return f(src, dst, A, C)


def kernel(x, edge_index, edge_attr, W_msg, b_msg, W_upd, b_upd):
    D = x.shape[1]
    W1 = W_msg[:D]
    W2 = W_msg[D:2 * D]
    W3 = W_msg[2 * D:]
    Wu1 = W_upd[:D]
    Wu2 = W_upd[D:]

    Wcat = jnp.concatenate([W1, W2, Wu1], axis=1)
    bcat = jnp.concatenate(
        [jnp.zeros((2 * OUTD,), jnp.float32), b_upd]
    )[None, :]

    A, B, XU = _node_pre(x, Wcat, bcat)

    C = _edge_pre(edge_attr, W3, b_msg[None, :])

    S = _segmax(edge_index[0], edge_index[1], A, C)

    return _final(S, B, XU, Wu2)


# speedup vs baseline: 1.0103x; 1.0103x over previous
import functools

import jax
import jax.numpy as jnp
from jax import lax
from jax.experimental import pallas as pl
from jax.experimental.pallas import tpu as pltpu
from jax.experimental.pallas import tpu_sc as plsc

NN = 10000
EE = 320000
DD = 128
DEE = 16
OUTD = 128

NC = 2
NS = 16
NW = NC * NS
NPW = 320
KB = 8000
NBLK = EE // KB
GRP = 64
NSPL = 4
STG = 80


def _node_pre_body(x_ref, w_ref, b_ref, a_ref, b2_ref, xu_ref):
    abx = (
        jnp.dot(x_ref[...], w_ref[...], preferred_element_type=jnp.float32)
        + b_ref[...]
    )
    a_ref[...] = abx[:, :OUTD]
    b2_ref[...] = abx[:, OUTD:2 * OUTD]
    xu_ref[...] = abx[:, 2 * OUTD:]


def _node_pre(x, Wcat, bcat):
    sds = jax.ShapeDtypeStruct((NN, OUTD), jnp.float32)
    return pl.pallas_call(
        _node_pre_body,
        out_shape=[sds, sds, sds],
    )(x, Wcat, bcat)


def _edge_pre_body(e_ref, w_ref, b_ref, o_ref):
    o_ref[...] = (
        jnp.dot(e_ref[...], w_ref[...], preferred_element_type=jnp.float32)
        + b_ref[...]
    )


def _edge_pre(edge_attr, W3, b_msg):
    RB = 20000
    grid = EE // RB
    return pl.pallas_call(
        _edge_pre_body,
        grid=(grid,),
        in_specs=[
            pl.BlockSpec((RB, DEE), lambda i: (i, 0)),
            pl.BlockSpec((DEE, OUTD), lambda i: (0, 0)),
            pl.BlockSpec((1, OUTD), lambda i: (0, 0)),
        ],
        out_specs=pl.BlockSpec((RB, OUTD), lambda i: (i, 0)),
        out_shape=jax.ShapeDtypeStruct((EE, OUTD), jnp.float32),
    )(edge_attr, W3, b_msg)


def _final_body(s_ref, b_ref, xu_ref, w_ref, o_ref):
    agg = jnp.maximum(s_ref[...] + b_ref[...], 0.0)
    o_ref[...] = xu_ref[...] + jnp.dot(
        agg, w_ref[...], preferred_element_type=jnp.float32
    )


def _final(S, B, XU, Wu2):
    return pl.pallas_call(
        _final_body,
        out_shape=jax.ShapeDtypeStruct((NN, OUTD), jnp.float32),
    )(S, B, XU, Wu2)


def _segmax_body(src_hbm, dst_hbm, a_hbm, c_hbm, s_hbm,
                 acc0, acc1, acc2, acc3, acc4, acc5, acc6, acc7,
                 dblk, sblk, mbuf, sidx, dlbuf, arows, crows, stage,
                 sem_a, sem_c):
    wid = lax.axis_index("s") * NC + lax.axis_index("c")
    lo = wid * NPW
    hi = lo + NPW
    accs = [acc0, acc1, acc2, acc3, acc4, acc5, acc6, acc7]

    ninf = jnp.full((16,), -jnp.inf, jnp.float32)
    zeros = jnp.zeros((16,), jnp.int32)
    ones = jnp.ones((16,), jnp.int32)
    lanes = lax.iota(jnp.int32, 16)

    def _init_acc(r, _):
        for f in range(OUTD // 16):
            accs[f][pl.ds(r * 16, 16)] = ninf
        return 0
    lax.fori_loop(0, NPW + 1, _init_acc, 0)

    def _init_mbuf(k, _):
        mbuf[pl.ds(k * 16, 16)] = zeros
        return 0
    lax.fori_loop(0, (KB + GRP) // 16, _init_mbuf, 0)

    def _block(b, _):
        base = b * KB
        pltpu.sync_copy(dst_hbm.at[pl.ds(base, KB)], dblk)
        pltpu.sync_copy(src_hbm.at[pl.ds(base, KB)], sblk)

        def _scan(i, offv):
            d = dblk[pl.ds(i * 16, 16)]
            m = (d >= lo) & (d < hi)
            mi = jnp.where(m, ones, zeros)
            cntv = plsc.all_reduce_population_count(m)
            pos = offv + plsc.cumsum(mi) - 1
            pos = jnp.where(m, pos, 0)
            eids = base + i * 16 + lanes
            plsc.store_scatter(mbuf, [pos], eids, mask=m)
            return offv + cntv

        offv = lax.fori_loop(0, KB // 16, _scan, zeros, unroll=8)
        m_cnt = offv[0]

        ngrp = (m_cnt + GRP - 1) // GRP

        def _issue(g, par):
            goff = g * GRP
            rem = m_cnt - goff

            def _pre(s, _):
                mv = mbuf[pl.ds(goff + s * 16, 16)]
                valid = (lanes + s * 16) < rem
                lid = jnp.where(valid, mv - base, 0)
                dstv = plsc.load_gather(dblk, [lid])
                srcv = plsc.load_gather(sblk, [lid])
                dlbuf[par, pl.ds(s * 16, 16)] = jnp.where(valid, dstv - lo, NPW)
                sidx[par, pl.ds(s * 16, 16)] = srcv
                return 0
            lax.fori_loop(0, GRP // 16, _pre, 0, unroll=True)

            q = GRP // NSPL
            for u in range(NSPL):
                pltpu.async_copy(
                    c_hbm.at[mbuf.at[pl.ds(goff + u * q, q)]],
                    crows.at[pl.ds(par * GRP + u * q, q)], sem_c.at[par])
                pltpu.async_copy(
                    a_hbm.at[sidx.at[par, pl.ds(u * q, q)]],
                    arows.at[pl.ds(par * GRP + u * q, q)], sem_a.at[par])

        def _process(g, par):
            goff = g * GRP
            q = GRP // NSPL
            for u in range(NSPL):
                pltpu.make_async_copy(
                    a_hbm.at[sidx.at[par, pl.ds(u * q, q)]],
                    arows.at[pl.ds(par * GRP + u * q, q)], sem_a.at[par]).wait()
                pltpu.make_async_copy(
                    c_hbm.at[mbuf.at[pl.ds(goff + u * q, q)]],
                    crows.at[pl.ds(par * GRP + u * q, q)], sem_c.at[par]).wait()

            for s in range(GRP // 16):
                dlv = dlbuf[par, pl.ds(s * 16, 16)]
                for j in range(16):
                    e = par * GRP + s * 16 + j
                    dspl = dlv.at[jnp.full((16,), j, jnp.int32)].get(
                        mode="promise_in_bounds")
                    fidx = dspl * 16 + lanes
                    nf = OUTD // 16
                    avs = [arows[e, pl.ds(f * 16, 16)] for f in range(nf)]
                    cvs = [crows[e, pl.ds(f * 16, 16)] for f in range(nf)]
                    gvs = [plsc.load_gather(accs[f], [fidx])
                           for f in range(nf)]
                    nvs = [jnp.maximum(gvs[f], avs[f] + cvs[f])
                           for f in range(nf)]
                    for f in range(nf):
                        plsc.store_scatter(accs[f], [fidx], nvs[f])

        @pl.when(ngrp > 0)
        def _():
            _issue(0, 0)

        def _gloop(g, _):
            par = g % 2

            @pl.when(g + 1 < ngrp)
            def _():
                _issue(g + 1, 1 - par)

            _process(g, par)
            return 0

        lax.fori_loop(0, ngrp, _gloop, 0)
        return 0

    lax.fori_loop(0, NBLK, _block, 0)

    nchunks = jnp.where(wid < NW - 1, NPW // STG, 1)

    def _wchunk(k, _):
        def _wrow(r, _2):
            for f in range(OUTD // 16):
                stage[r, pl.ds(f * 16, 16)] = accs[f][pl.ds((k * STG + r) * 16, 16)]
            return 0
        lax.fori_loop(0, STG, _wrow, 0)
        pltpu.sync_copy(stage, s_hbm.at[pl.ds(lo + k * STG, STG)])
        return 0

    lax.fori_loop(0, nchunks, _wchunk, 0)


def _segmax(src, dst, A, C):
    mesh = plsc.VectorSubcoreMesh(core_axis_name="c", subcore_axis_name="s")
    f = functools.partial(
        pl.kernel,
        out_type=jax.ShapeDtypeStruct((NN, OUTD), jnp.float32),
        mesh=mesh,
        compiler_params=pltpu.CompilerParams(needs_layout_passes=False),
        scratch_types=[
            pltpu.VMEM(((NPW + 1) * 16,), jnp.float32),
            pltpu.VMEM(((NPW + 1) * 16,), jnp.float32),
            pltpu.VMEM(((NPW + 1) * 16,), jnp.float32),
            pltpu.VMEM(((NPW + 1) * 16,), jnp.float32),
            pltpu.VMEM(((NPW + 1) * 16,), jnp.float32),
            pltpu.VMEM(((NPW + 1) * 16,), jnp.float32),
            pltpu.VMEM(((NPW + 1) * 16,), jnp.float32),
            pltpu.VMEM(((NPW + 1) * 16,), jnp.float32),
            pltpu.VMEM((KB,), jnp.int32),
            pltpu.VMEM((KB,), jnp.int32),
            pltpu.VMEM((KB + GRP,), jnp.int32),
            pltpu.VMEM((2, GRP), jnp.int32),
            pltpu.VMEM((2, GRP), jnp.int32),
            pltpu.VMEM((2 * GRP, OUTD), jnp.float32),
            pltpu.VMEM((2 * GRP, OUTD), jnp.float32),
            pltpu.VMEM((STG, OUTD), jnp.float32),
            pltpu.SemaphoreType.DMA((2,)),
            pltpu.SemaphoreType.DMA((2,)),
        ],
    )(_segmax_body)
    return f(src, dst, A, C)


def kernel(x, edge_index, edge_attr, W_msg, b_msg, W_upd, b_upd):
    D = x.shape[1]
    W1 = W_msg[:D]
    W2 = W_msg[D:2 * D]
    W3 = W_msg[2 * D:]
    Wu1 = W_upd[:D]
    Wu2 = W_upd[D:]

    Wcat = jnp.concatenate([W1, W2, Wu1], axis=1)
    bcat = jnp.concatenate(
        [jnp.zeros((2 * OUTD,), jnp.float32), b_upd]
    )[None, :]

    A, B, XU = _node_pre(x, Wcat, bcat)

    C = _edge_pre(edge_attr, W3, b_msg[None, :])

    S = _segmax(edge_index[0], edge_index[1], A, C)

    return _final(S, B, XU, Wu2)


# speedup vs baseline: 1.0771x; 1.0661x over previous
import functools

import jax
import jax.numpy as jnp
from jax import lax
from jax.experimental import pallas as pl
from jax.experimental.pallas import tpu as pltpu
from jax.experimental.pallas import tpu_sc as plsc

NN = 10000
EE = 320000
DD = 128
DEE = 16
OUTD = 128

NC = 2
NS = 16
NW = NC * NS
NPW = 320
KB = 10000
NBLK = EE // KB
GRP = 64
NSPL = 4
STG = 80


def _node_pre_body(x_ref, w_ref, b_ref, a_ref, b2_ref, xu_ref):
    abx = (
        jnp.dot(x_ref[...], w_ref[...], preferred_element_type=jnp.float32)
        + b_ref[...]
    )
    a_ref[...] = abx[:, :OUTD]
    b2_ref[...] = abx[:, OUTD:2 * OUTD]
    xu_ref[...] = abx[:, 2 * OUTD:]


def _node_pre(x, Wcat, bcat):
    sds = jax.ShapeDtypeStruct((NN, OUTD), jnp.float32)
    return pl.pallas_call(
        _node_pre_body,
        out_shape=[sds, sds, sds],
    )(x, Wcat, bcat)


def _edge_pre_body(e_ref, w_ref, b_ref, o_ref):
    o_ref[...] = (
        jnp.dot(e_ref[...], w_ref[...], preferred_element_type=jnp.float32)
        + b_ref[...]
    )


def _edge_pre(edge_attr, W3, b_msg):
    RB = 20000
    grid = EE // RB
    return pl.pallas_call(
        _edge_pre_body,
        grid=(grid,),
        in_specs=[
            pl.BlockSpec((RB, DEE), lambda i: (i, 0)),
            pl.BlockSpec((DEE, OUTD), lambda i: (0, 0)),
            pl.BlockSpec((1, OUTD), lambda i: (0, 0)),
        ],
        out_specs=pl.BlockSpec((RB, OUTD), lambda i: (i, 0)),
        out_shape=jax.ShapeDtypeStruct((EE, OUTD), jnp.float32),
    )(edge_attr, W3, b_msg)


def _final_body(s_ref, b_ref, xu_ref, w_ref, o_ref):
    agg = jnp.maximum(s_ref[...] + b_ref[...], 0.0)
    o_ref[...] = xu_ref[...] + jnp.dot(
        agg, w_ref[...], preferred_element_type=jnp.float32
    )


def _final(S, B, XU, Wu2):
    return pl.pallas_call(
        _final_body,
        out_shape=jax.ShapeDtypeStruct((NN, OUTD), jnp.float32),
    )(S, B, XU, Wu2)


def _segmax_body(src_hbm, dst_hbm, a_hbm, c_hbm, s_hbm,
                 acc0, acc1, acc2, acc3, acc4, acc5, acc6, acc7,
                 dblk, sblk, mbuf, sidx, dlbuf, arows, crows, stage,
                 sem_a, sem_c):
    wid = lax.axis_index("s") * NC + lax.axis_index("c")
    lo = wid * NPW
    hi = lo + NPW
    accs = [acc0, acc1, acc2, acc3, acc4, acc5, acc6, acc7]

    ninf = jnp.full((16,), -jnp.inf, jnp.float32)
    zeros = jnp.zeros((16,), jnp.int32)
    ones = jnp.ones((16,), jnp.int32)
    lanes = lax.iota(jnp.int32, 16)

    def _init_acc(r, _):
        for f in range(OUTD // 16):
            accs[f][pl.ds(r * 16, 16)] = ninf
        return 0
    lax.fori_loop(0, NPW + 1, _init_acc, 0)

    def _init_mbuf(k, _):
        mbuf[pl.ds(k * 16, 16)] = zeros
        return 0
    lax.fori_loop(0, (KB + GRP) // 16, _init_mbuf, 0)

    def _block(b, _):
        base = b * KB
        pltpu.sync_copy(dst_hbm.at[pl.ds(base, KB)], dblk)
        pltpu.sync_copy(src_hbm.at[pl.ds(base, KB)], sblk)

        def _scan(i, offv):
            d = dblk[pl.ds(i * 16, 16)]
            m = (d >= lo) & (d < hi)
            mi = jnp.where(m, ones, zeros)
            cntv = plsc.all_reduce_population_count(m)
            pos = offv + plsc.cumsum(mi) - 1
            pos = jnp.where(m, pos, 0)
            eids = base + i * 16 + lanes
            plsc.store_scatter(mbuf, [pos], eids, mask=m)
            return offv + cntv

        offv = lax.fori_loop(0, KB // 16, _scan, zeros, unroll=8)
        m_cnt = offv[0]

        ngrp = (m_cnt + GRP - 1) // GRP

        def _issue(g, par):
            goff = g * GRP
            rem = m_cnt - goff

            def _pre(s, _):
                mv = mbuf[pl.ds(goff + s * 16, 16)]
                valid = (lanes + s * 16) < rem
                lid = jnp.where(valid, mv - base, 0)
                dstv = plsc.load_gather(dblk, [lid])
                srcv = plsc.load_gather(sblk, [lid])
                dlbuf[par, pl.ds(s * 16, 16)] = jnp.where(valid, dstv - lo, NPW)
                sidx[par, pl.ds(s * 16, 16)] = srcv
                return 0
            lax.fori_loop(0, GRP // 16, _pre, 0, unroll=True)

            q = GRP // NSPL
            for u in range(NSPL):
                pltpu.async_copy(
                    c_hbm.at[mbuf.at[pl.ds(goff + u * q, q)]],
                    crows.at[pl.ds(par * GRP + u * q, q)], sem_c.at[par])
                pltpu.async_copy(
                    a_hbm.at[sidx.at[par, pl.ds(u * q, q)]],
                    arows.at[pl.ds(par * GRP + u * q, q)], sem_a.at[par])

        def _process(g, par):
            goff = g * GRP
            q = GRP // NSPL
            for u in range(NSPL):
                pltpu.make_async_copy(
                    a_hbm.at[sidx.at[par, pl.ds(u * q, q)]],
                    arows.at[pl.ds(par * GRP + u * q, q)], sem_a.at[par]).wait()
                pltpu.make_async_copy(
                    c_hbm.at[mbuf.at[pl.ds(goff + u * q, q)]],
                    crows.at[pl.ds(par * GRP + u * q, q)], sem_c.at[par]).wait()

            for s in range(GRP // 16):
                dlv = dlbuf[par, pl.ds(s * 16, 16)]
                for j in range(16):
                    e = par * GRP + s * 16 + j
                    dspl = dlv.at[jnp.full((16,), j, jnp.int32)].get(
                        mode="promise_in_bounds")
                    fidx = dspl * 16 + lanes
                    nf = OUTD // 16
                    avs = [arows[e, pl.ds(f * 16, 16)] for f in range(nf)]
                    cvs = [crows[e, pl.ds(f * 16, 16)] for f in range(nf)]
                    gvs = [plsc.load_gather(accs[f], [fidx])
                           for f in range(nf)]
                    nvs = [jnp.maximum(gvs[f], avs[f] + cvs[f])
                           for f in range(nf)]
                    for f in range(nf):
                        plsc.store_scatter(accs[f], [fidx], nvs[f])

        @pl.when(ngrp > 0)
        def _():
            _issue(0, 0)

        def _gloop(g, _):
            par = g % 2

            @pl.when(g + 1 < ngrp)
            def _():
                _issue(g + 1, 1 - par)

            _process(g, par)
            return 0

        lax.fori_loop(0, ngrp, _gloop, 0)
        return 0

    lax.fori_loop(0, NBLK, _block, 0)

    nchunks = jnp.where(wid < NW - 1, NPW // STG, 1)

    def _wchunk(k, _):
        def _wrow(r, _2):
            for f in range(OUTD // 16):
                stage[r, pl.ds(f * 16, 16)] = accs[f][pl.ds((k * STG + r) * 16, 16)]
            return 0
        lax.fori_loop(0, STG, _wrow, 0)
        pltpu.sync_copy(stage, s_hbm.at[pl.ds(lo + k * STG, STG)])
        return 0

    lax.fori_loop(0, nchunks, _wchunk, 0)


def _segmax(src, dst, A, C):
    mesh = plsc.VectorSubcoreMesh(core_axis_name="c", subcore_axis_name="s")
    f = functools.partial(
        pl.kernel,
        out_type=jax.ShapeDtypeStruct((NN, OUTD), jnp.float32),
        mesh=mesh,
        compiler_params=pltpu.CompilerParams(needs_layout_passes=False),
        scratch_types=[
            pltpu.VMEM(((NPW + 1) * 16,), jnp.float32),
            pltpu.VMEM(((NPW + 1) * 16,), jnp.float32),
            pltpu.VMEM(((NPW + 1) * 16,), jnp.float32),
            pltpu.VMEM(((NPW + 1) * 16,), jnp.float32),
            pltpu.VMEM(((NPW + 1) * 16,), jnp.float32),
            pltpu.VMEM(((NPW + 1) * 16,), jnp.float32),
            pltpu.VMEM(((NPW + 1) * 16,), jnp.float32),
            pltpu.VMEM(((NPW + 1) * 16,), jnp.float32),
            pltpu.VMEM((KB,), jnp.int32),
            pltpu.VMEM((KB,), jnp.int32),
            pltpu.VMEM((KB + GRP,), jnp.int32),
            pltpu.VMEM((2, GRP), jnp.int32),
            pltpu.VMEM((2, GRP), jnp.int32),
            pltpu.VMEM((2 * GRP, OUTD), jnp.float32),
            pltpu.VMEM((2 * GRP, OUTD), jnp.float32),
            pltpu.VMEM((STG, OUTD), jnp.float32),
            pltpu.SemaphoreType.DMA((2,)),
            pltpu.SemaphoreType.DMA((2,)),
        ],
    )(_segmax_body)
    return f(src, dst, A, C)


def kernel(x, edge_index, edge_attr, W_msg, b_msg, W_upd, b_upd):
    D = x.shape[1]
    W1 = W_msg[:D]
    W2 = W_msg[D:2 * D]
    W3 = W_msg[2 * D:]
    Wu1 = W_upd[:D]
    Wu2 = W_upd[D:]

    Wcat = jnp.concatenate([W1, W2, Wu1], axis=1)
    bcat = jnp.concatenate(
        [jnp.zeros((2 * OUTD,), jnp.float32), b_upd]
    )[None, :]

    A, B, XU = _node_pre(x, Wcat, bcat)

    C = _edge_pre(edge_attr, W3, b_msg[None, :])

    S = _segmax(edge_index[0], edge_index[1], A, C)

    return _final(S, B, XU, Wu2)
